# Initial kernel scaffold; baseline (speedup 1.0000x reference)
import jax
import jax.numpy as jnp
from jax.experimental import pallas as pl


def kernel(x, edge_index):
    raise NotImplementedError("write your pallas kernel here")


# speedup vs baseline: 14.0528x; 14.0528x over previous
import functools

import jax
import jax.numpy as jnp
from jax import lax
from jax.experimental import pallas as pl
from jax.experimental.pallas import tpu as pltpu
from jax.experimental.pallas import tpu_sc as plsc

_N = 10000
_E = 320000
_D = 128
_NC = 2
_NS = 16
_NW = _NC * _NS
_EPW = _E // _NW
_CHUNK = 80
_NCHUNK = _EPW // _CHUNK
_NP = 10240
_RPT = _NP // _NS
_HR, _HC = 64, 160

_mesh = plsc.VectorSubcoreMesh(core_axis_name="c", subcore_axis_name="s")


@functools.partial(
    pl.kernel,
    mesh=_mesh,
    out_type=jax.ShapeDtypeStruct((_NW, _HR, _HC), jnp.float32),
    compiler_params=pltpu.CompilerParams(use_tc_tiling_on_sc=False,
                                         needs_layout_passes=False),
    scratch_types=[
        pltpu.VMEM((_EPW,), jnp.int32),
        pltpu.VMEM((_HR, _HC), jnp.float32),
    ],
)
def _deg_kernel(dst_hbm, out_hbm, dst_v, hist_v):
  cid = lax.axis_index("c")
  sid = lax.axis_index("s")
  wid = sid * _NC + cid

  zeros = jnp.zeros((16,), jnp.float32)

  def zbody(i, carry):
    r = i // (_HC // 16)
    c = i % (_HC // 16)
    hist_v[r, pl.ds(c * 16, 16)] = zeros
    return carry

  lax.fori_loop(0, _HR * (_HC // 16), zbody, 0)

  pltpu.sync_copy(dst_hbm.at[pl.ds(wid * _EPW, _EPW)], dst_v)

  ones = jnp.ones((16,), jnp.float32)

  def body(g, carry):
    idx = dst_v[pl.ds(g * 16, 16)]
    plsc.addupdate_scatter(hist_v, [idx // _HC, idx % _HC], ones)
    return carry

  lax.fori_loop(0, _EPW // 16, body, 0)
  pltpu.sync_copy(hist_v, out_hbm.at[wid])


def _make_agg(width):
  @functools.partial(
      pl.kernel,
      mesh=_mesh,
      out_type=jax.ShapeDtypeStruct((_NC, _NP, width), jnp.float32),
      scratch_types=[
          pltpu.VMEM((_CHUNK,), jnp.int32),
          pltpu.VMEM((_CHUNK,), jnp.int32),
          pltpu.VMEM((_CHUNK, width), jnp.float32),
          pltpu.VMEM_SHARED((_NP, width), jnp.float32),
          pltpu.SemaphoreType.DMA,
      ],
  )
  def agg(table_hbm, src_hbm, dst_hbm, zero_hbm, out_hbm,
          src_v, dst_v, rows_v, acc_sh, sem):
    cid = lax.axis_index("c")
    sid = lax.axis_index("s")
    wid = sid * _NC + cid
    pltpu.sync_copy(zero_hbm, acc_sh.at[pl.ds(sid * _RPT, _RPT)])
    plsc.subcore_barrier()

    ebase = wid * _EPW

    def body(i, carry):
      off = ebase + i * _CHUNK
      pltpu.sync_copy(src_hbm.at[pl.ds(off, _CHUNK)], src_v)
      pltpu.async_copy(table_hbm.at[src_v], rows_v, sem).wait()
      pltpu.sync_copy(dst_hbm.at[pl.ds(off, _CHUNK)], dst_v)
      pltpu.sync_copy(rows_v, acc_sh.at[dst_v], add=True)
      return carry

    lax.fori_loop(0, _NCHUNK, body, 0)
    plsc.subcore_barrier()
    pltpu.sync_copy(acc_sh.at[pl.ds(sid * _RPT, _RPT)],
                    out_hbm.at[cid, pl.ds(sid * _RPT, _RPT)])

  return agg


_agg128 = _make_agg(_D)


def _scale_body(power, with_partials):
  def body(degt_ref, base_ref, *rest):
    if with_partials:
      part_ref, o_ref = rest
    else:
      (o_ref,) = rest
    deg = jnp.sum(degt_ref[...], axis=1, keepdims=True) + 1.0
    dis = lax.rsqrt(deg)
    s = dis * dis if power == 2 else dis
    acc = base_ref[...]
    if with_partials:
      acc = acc + part_ref[0] + part_ref[1]
    o_ref[...] = s * acc
  return body


_BN = 2000


def _scale(degt, base, partials, power):
  grid = (_N // _BN,)
  in_specs = [
      pl.BlockSpec((_BN, _NW), lambda i: (i, 0)),
      pl.BlockSpec((_BN, _D), lambda i: (i, 0)),
  ]
  args = [degt, base]
  if partials is not None:
    in_specs.append(pl.BlockSpec((_NC, _BN, _D), lambda i: (0, i, 0)))
    args.append(partials)
  return pl.pallas_call(
      _scale_body(power, partials is not None),
      grid=grid,
      in_specs=in_specs,
      out_specs=pl.BlockSpec((_BN, _D), lambda i: (i, 0)),
      out_shape=jax.ShapeDtypeStruct((_N, _D), jnp.float32),
  )(*args)


def kernel(x, edge_index):
  src = edge_index[0].astype(jnp.int32)
  dst = edge_index[1].astype(jnp.int32)
  zeros128 = jnp.zeros((_RPT, _D), jnp.float32)

  degp = _deg_kernel(dst)
  degt = degp.reshape(_NW, _NP).T
  z1 = _scale(degt, x, None, power=1)
  a1 = _agg128(z1, src, dst, zeros128)
  z2 = _scale(degt, z1, a1, power=2)
  a2 = _agg128(z2, src, dst, zeros128)
  return _scale(degt, z2, a2, power=1)


# speedup vs baseline: 31.5974x; 2.2485x over previous
import functools

import jax
import jax.numpy as jnp
from jax import lax
from jax.experimental import pallas as pl
from jax.experimental.pallas import tpu as pltpu
from jax.experimental.pallas import tpu_sc as plsc

_N = 10000
_E = 320000
_D = 128
_NC = 2
_NS = 16
_NW = _NC * _NS
_EPW = _E // _NW
_CHUNK = 40
_NCHUNK = _EPW // _CHUNK
_NP = 10240
_RPT = _NP // _NS
_HR, _HC = 64, 160

_mesh = plsc.VectorSubcoreMesh(core_axis_name="c", subcore_axis_name="s")


@functools.partial(
    pl.kernel,
    mesh=_mesh,
    out_type=jax.ShapeDtypeStruct((_NW, _HR, _HC), jnp.float32),
    compiler_params=pltpu.CompilerParams(use_tc_tiling_on_sc=False,
                                         needs_layout_passes=False),
    scratch_types=[
        pltpu.VMEM((_EPW,), jnp.int32),
        pltpu.VMEM((_HR, _HC), jnp.float32),
    ],
)
def _deg_kernel(dst_hbm, out_hbm, dst_v, hist_v):
  cid = lax.axis_index("c")
  sid = lax.axis_index("s")
  wid = sid * _NC + cid

  zeros = jnp.zeros((16,), jnp.float32)

  def zbody(i, carry):
    r = i // (_HC // 16)
    c = i % (_HC // 16)
    hist_v[r, pl.ds(c * 16, 16)] = zeros
    return carry

  lax.fori_loop(0, _HR * (_HC // 16), zbody, 0)

  pltpu.sync_copy(dst_hbm.at[pl.ds(wid * _EPW, _EPW)], dst_v)

  ones = jnp.ones((16,), jnp.float32)

  def body(g, carry):
    idx = dst_v[pl.ds(g * 16, 16)]
    plsc.addupdate_scatter(hist_v, [idx // _HC, idx % _HC], ones)
    return carry

  lax.fori_loop(0, _EPW // 16, body, 0)
  pltpu.sync_copy(hist_v, out_hbm.at[wid])


_NBUF = 5
_PREF = 2


def _make_agg(width):
  @functools.partial(
      pl.kernel,
      mesh=_mesh,
      out_type=jax.ShapeDtypeStruct((_NC, _NP, width), jnp.float32),
      scratch_types=[
          pltpu.VMEM((_EPW,), jnp.int32),
      ] + [pltpu.VMEM((_CHUNK,), jnp.int32) for _ in range(_NBUF)] + [
          pltpu.VMEM((_CHUNK, width), jnp.float32) for _ in range(_NBUF)] + [
          pltpu.VMEM_SHARED((_NP, width), jnp.float32),
          pltpu.SemaphoreType.DMA((_NBUF,)),
          pltpu.SemaphoreType.DMA((_NBUF,)),
          pltpu.SemaphoreType.DMA((_NBUF,)),
      ],
  )
  def agg(table_hbm, src_hbm, dst_hbm, zero_hbm, out_hbm, srcall_v, *rest):
    dst_v = rest[:_NBUF]
    rows_v = rest[_NBUF:2 * _NBUF]
    acc_sh, gsem, ssem, dsem = rest[2 * _NBUF:]
    cid = lax.axis_index("c")
    sid = lax.axis_index("s")
    wid = sid * _NC + cid
    ebase = wid * _EPW
    pltpu.sync_copy(zero_hbm, acc_sh.at[pl.ds(sid * _RPT, _RPT)])
    pltpu.sync_copy(src_hbm.at[pl.ds(ebase, _EPW)], srcall_v)
    plsc.subcore_barrier()

    def gather_of(c, b):
      return pltpu.make_async_copy(
          table_hbm.at[srcall_v.at[pl.ds(c * _CHUNK, _CHUNK)]],
          rows_v[b], gsem.at[b])

    def dstload_of(c, b):
      return pltpu.make_async_copy(
          dst_hbm.at[pl.ds(ebase + c * _CHUNK, _CHUNK)], dst_v[b], dsem.at[b])

    def scatter_of(b):
      return pltpu.make_async_copy(rows_v[b], acc_sh.at[dst_v[b]], ssem.at[b])

    for b in range(_PREF):
      dstload_of(b, b).start()
      gather_of(b, b).start()

    def outer(i, carry):
      for b in range(_NBUF):
        c = i * _NBUF + b
        p = c + _PREF
        pb = (b + _PREF) % _NBUF

        @pl.when(jnp.logical_and(p < _NCHUNK, c >= _NBUF - _PREF))
        def _():
          scatter_of(pb).wait()

        @pl.when(p < _NCHUNK)
        def _():
          dstload_of(p, pb).start()
          gather_of(p, pb).start()

        gather_of(c, b).wait()
        dstload_of(c, b).wait()
        pltpu.async_copy(rows_v[b], acc_sh.at[dst_v[b]], ssem.at[b], add=True)
      return carry

    lax.fori_loop(0, _NCHUNK // _NBUF, outer, 0)
    for b in range(_NBUF):
      scatter_of(b).wait()
    plsc.subcore_barrier()
    pltpu.sync_copy(acc_sh.at[pl.ds(sid * _RPT, _RPT)],
                    out_hbm.at[cid, pl.ds(sid * _RPT, _RPT)])

  return agg


_agg128 = _make_agg(_D)


def _scale_body(power, with_partials):
  def body(degt_ref, base_ref, *rest):
    if with_partials:
      part_ref, o_ref = rest
    else:
      (o_ref,) = rest
    deg = jnp.sum(degt_ref[...], axis=1, keepdims=True) + 1.0
    dis = lax.rsqrt(deg)
    s = dis * dis if power == 2 else dis
    acc = base_ref[...]
    if with_partials:
      acc = acc + part_ref[0] + part_ref[1]
    o_ref[...] = s * acc
  return body


_BN = 2000


def _scale(degt, base, partials, power):
  grid = (_N // _BN,)
  in_specs = [
      pl.BlockSpec((_BN, _NW), lambda i: (i, 0)),
      pl.BlockSpec((_BN, _D), lambda i: (i, 0)),
  ]
  args = [degt, base]
  if partials is not None:
    in_specs.append(pl.BlockSpec((_NC, _BN, _D), lambda i: (0, i, 0)))
    args.append(partials)
  return pl.pallas_call(
      _scale_body(power, partials is not None),
      grid=grid,
      in_specs=in_specs,
      out_specs=pl.BlockSpec((_BN, _D), lambda i: (i, 0)),
      out_shape=jax.ShapeDtypeStruct((_N, _D), jnp.float32),
  )(*args)


def kernel(x, edge_index):
  src = edge_index[0].astype(jnp.int32)
  dst = edge_index[1].astype(jnp.int32)
  zeros128 = jnp.zeros((_RPT, _D), jnp.float32)

  degp = _deg_kernel(dst)
  degt = degp.reshape(_NW, _NP).T
  z1 = _scale(degt, x, None, power=1)
  a1 = _agg128(z1, src, dst, zeros128)
  z2 = _scale(degt, z1, a1, power=2)
  a2 = _agg128(z2, src, dst, zeros128)
  return _scale(degt, z2, a2, power=1)
